# baseline (speedup 1.0000x reference)
import jax
import jax.numpy as jnp
from jax.experimental import pallas as pl
from jax.experimental.pallas import tpu as pltpu

_BLOCK = 20000
_PE_PAD = 32


def _fused_kernel(x_ref, ids_ref, w_ref, b_ref, pe_ref, out_ref):
    x_blk = x_ref[...]
    ids = ids_ref[0, 0, :]
    h = jnp.dot(x_blk, w_ref[...], preferred_element_type=jnp.float32)
    h = h + b_ref[0, :]
    onehot = (ids[:, None] == jax.lax.broadcasted_iota(
        jnp.int32, (ids.shape[0], _PE_PAD), 1)).astype(jnp.float32)
    pos = jnp.dot(onehot, pe_ref[...], preferred_element_type=jnp.float32)
    out_ref[...] = jnp.concatenate([h, pos], axis=-1)


def kernel(x, batch, W, b, pe_table):
    n, dim_in = x.shape
    d_out = W.shape[1]
    dim_pe = pe_table.shape[1]
    nb = n // _BLOCK
    ids3 = batch.astype(jnp.int32).reshape(nb, 1, _BLOCK)
    b2 = b.reshape(1, d_out)
    pe_pad = jnp.zeros((_PE_PAD, dim_pe), jnp.float32).at[:pe_table.shape[0]].set(pe_table)

    return pl.pallas_call(
        _fused_kernel,
        grid=(nb,),
        in_specs=[
            pl.BlockSpec((_BLOCK, dim_in), lambda i: (i, 0)),
            pl.BlockSpec((1, 1, _BLOCK), lambda i: (i, 0, 0)),
            pl.BlockSpec((dim_in, d_out), lambda i: (0, 0)),
            pl.BlockSpec((1, d_out), lambda i: (0, 0)),
            pl.BlockSpec((_PE_PAD, dim_pe), lambda i: (0, 0)),
        ],
        out_specs=pl.BlockSpec((_BLOCK, d_out + dim_pe), lambda i: (i, 0)),
        out_shape=jax.ShapeDtypeStruct((n, d_out + dim_pe), jnp.float32),
        compiler_params=pltpu.CompilerParams(
            dimension_semantics=("arbitrary",),
        ),
    )(x, ids3, W, b2, pe_pad)


# speedup vs baseline: 1.0088x; 1.0088x over previous
import jax
import jax.numpy as jnp
from jax.experimental import pallas as pl
from jax.experimental.pallas import tpu as pltpu

_BLOCK = 10000
_PE_PAD = 32


def _fused_kernel(x_ref, ids_ref, w_ref, b_ref, pe_ref, out_ref):
    x_blk = x_ref[...]
    ids = ids_ref[0, 0, :]
    h = jnp.dot(x_blk, w_ref[...], preferred_element_type=jnp.float32)
    h = h + b_ref[0, :]
    onehot = (ids[:, None] == jax.lax.broadcasted_iota(
        jnp.int32, (ids.shape[0], _PE_PAD), 1)).astype(jnp.float32)
    pos = jnp.dot(onehot, pe_ref[...], preferred_element_type=jnp.float32)
    out_ref[...] = jnp.concatenate([h, pos], axis=-1)


def kernel(x, batch, W, b, pe_table):
    n, dim_in = x.shape
    d_out = W.shape[1]
    dim_pe = pe_table.shape[1]
    nb = n // _BLOCK
    ids3 = batch.astype(jnp.int32).reshape(nb, 1, _BLOCK)
    b2 = b.reshape(1, d_out)
    pe_pad = jnp.zeros((_PE_PAD, dim_pe), jnp.float32).at[:pe_table.shape[0]].set(pe_table)

    return pl.pallas_call(
        _fused_kernel,
        grid=(nb,),
        in_specs=[
            pl.BlockSpec((_BLOCK, dim_in), lambda i: (i, 0)),
            pl.BlockSpec((1, 1, _BLOCK), lambda i: (i, 0, 0)),
            pl.BlockSpec((dim_in, d_out), lambda i: (0, 0)),
            pl.BlockSpec((1, d_out), lambda i: (0, 0)),
            pl.BlockSpec((_PE_PAD, dim_pe), lambda i: (0, 0)),
        ],
        out_specs=pl.BlockSpec((_BLOCK, d_out + dim_pe), lambda i: (i, 0)),
        out_shape=jax.ShapeDtypeStruct((n, d_out + dim_pe), jnp.float32),
        compiler_params=pltpu.CompilerParams(
            dimension_semantics=("arbitrary",),
        ),
    )(x, ids3, W, b2, pe_pad)
